# baseline (speedup 1.0000x reference)
import functools

import jax
import jax.numpy as jnp
from jax import lax
from jax.experimental import pallas as pl
from jax.experimental.pallas import tpu as pltpu
from jax.experimental.pallas import tpu_sc as plsc

ROWS = 8192
SIZE = 1024
LANES = 16

_info = plsc.get_sparse_core_info()
NC = _info.num_cores
NS = _info.num_subcores
NW = NC * NS
ROWS_PER_W = ROWS // NW
BLK_ROWS = 16
NBLK = ROWS_PER_W // BLK_ROWS
BLK = BLK_ROWS * SIZE
CHUNKS = SIZE // LANES

_mesh = plsc.VectorSubcoreMesh(core_axis_name="c", subcore_axis_name="s")


@functools.partial(
    pl.kernel,
    mesh=_mesh,
    out_type=jax.ShapeDtypeStruct((ROWS * SIZE,), jnp.float32),
    scratch_types=[
        pltpu.VMEM((SIZE,), jnp.int32),
        pltpu.VMEM((BLK,), jnp.float32),
        pltpu.VMEM((BLK,), jnp.float32),
    ],
    compiler_params=pltpu.CompilerParams(needs_layout_passes=False),
)
def _permute_sc(z_hbm, p_hbm, out_hbm, p_v, zbuf, obuf):
    wid = lax.axis_index("s") * NC + lax.axis_index("c")
    base = wid * (ROWS_PER_W * SIZE)

    pltpu.sync_copy(p_hbm, p_v)

    def block_body(b, _):
        off = base + b * BLK
        pltpu.sync_copy(z_hbm.at[pl.ds(off, BLK)], zbuf)

        def jbody(j, _):
            col = j * LANES
            idx = p_v[pl.ds(col, LANES)]
            for r in range(BLK_ROWS):
                val = plsc.load_gather(zbuf, [idx])
                obuf[pl.ds(r * SIZE + col, LANES)] = val
                if r + 1 < BLK_ROWS:
                    idx = idx + SIZE
            return 0

        lax.fori_loop(0, CHUNKS, jbody, 0)
        pltpu.sync_copy(obuf, out_hbm.at[pl.ds(off, BLK)])
        return 0

    lax.fori_loop(0, NBLK, block_body, 0)


def kernel(z, p):
    zf = z.reshape(ROWS * SIZE)
    pi = p.astype(jnp.int32)
    out = _permute_sc(zf, pi)
    return out.reshape(ROWS, SIZE)


# speedup vs baseline: 1.1794x; 1.1794x over previous
import functools

import jax
import jax.numpy as jnp
from jax import lax
from jax.experimental import pallas as pl
from jax.experimental.pallas import tpu as pltpu
from jax.experimental.pallas import tpu_sc as plsc

ROWS = 8192
SIZE = 1024
LANES = 16

_info = plsc.get_sparse_core_info()
NC = _info.num_cores
NS = _info.num_subcores
NW = NC * NS
ROWS_PER_W = ROWS // NW
BLK_ROWS = 16
NBLK = ROWS_PER_W // BLK_ROWS
BLK = BLK_ROWS * SIZE
CHUNKS = SIZE // LANES

_mesh = plsc.VectorSubcoreMesh(core_axis_name="c", subcore_axis_name="s")


@functools.partial(
    pl.kernel,
    mesh=_mesh,
    out_type=jax.ShapeDtypeStruct((ROWS * SIZE,), jnp.float32),
    scratch_types=[
        pltpu.VMEM((SIZE,), jnp.int32),
        pltpu.VMEM((BLK,), jnp.float32),
        pltpu.VMEM((BLK,), jnp.float32),
        pltpu.VMEM((BLK,), jnp.float32),
        pltpu.VMEM((BLK,), jnp.float32),
        pltpu.SemaphoreType.DMA,
        pltpu.SemaphoreType.DMA,
        pltpu.SemaphoreType.DMA,
        pltpu.SemaphoreType.DMA,
    ],
    compiler_params=pltpu.CompilerParams(needs_layout_passes=False),
)
def _permute_sc(z_hbm, p_hbm, out_hbm, p_v, zbuf0, zbuf1, obuf0, obuf1,
                in_sem0, in_sem1, out_sem0, out_sem1):
    wid = lax.axis_index("s") * NC + lax.axis_index("c")
    base = wid * (ROWS_PER_W * SIZE)
    in_sems = (in_sem0, in_sem1)
    out_sems = (out_sem0, out_sem1)
    zbufs = (zbuf0, zbuf1)
    obufs = (obuf0, obuf1)

    pltpu.sync_copy(p_hbm, p_v)

    def start_in(b, s):
        pltpu.async_copy(z_hbm.at[pl.ds(base + b * BLK, BLK)], zbufs[s],
                         in_sems[s])

    def wait_in(b, s):
        pltpu.make_async_copy(z_hbm.at[pl.ds(base + b * BLK, BLK)],
                              zbufs[s], in_sems[s]).wait()

    def start_out(b, s):
        pltpu.async_copy(obufs[s], out_hbm.at[pl.ds(base + b * BLK, BLK)],
                         out_sems[s])

    def wait_out(b, s):
        pltpu.make_async_copy(obufs[s],
                              out_hbm.at[pl.ds(base + b * BLK, BLK)],
                              out_sems[s]).wait()

    def compute(s):
        zb = zbufs[s]
        ob = obufs[s]

        def jbody(j, _):
            col = j * LANES
            idx = p_v[pl.ds(col, LANES)]
            for r in range(BLK_ROWS):
                val = plsc.load_gather(zb, [idx])
                ob[pl.ds(r * SIZE + col, LANES)] = val
                if r + 1 < BLK_ROWS:
                    idx = idx + SIZE
            return 0

        lax.fori_loop(0, CHUNKS, jbody, 0)

    start_in(0, 0)
    start_in(1, 1)

    def ring(i, _):
        g = i * 2
        for s in range(2):
            b = g + s
            wait_in(b, s)

            @pl.when(i > 0)
            def _():
                wait_out(b - 2, s)

            compute(s)
            start_out(b, s)

            @pl.when(b + 2 < NBLK)
            def _():
                start_in(b + 2, s)

        return 0

    lax.fori_loop(0, NBLK // 2, ring, 0)
    wait_out(NBLK - 2, 0)
    wait_out(NBLK - 1, 1)


def kernel(z, p):
    zf = z.reshape(ROWS * SIZE)
    pi = p.astype(jnp.int32)
    out = _permute_sc(zf, pi)
    return out.reshape(ROWS, SIZE)


# speedup vs baseline: 1.9139x; 1.6227x over previous
import functools

import jax
import jax.numpy as jnp
from jax import lax
from jax.experimental import pallas as pl
from jax.experimental.pallas import tpu as pltpu
from jax.experimental.pallas import tpu_sc as plsc

ROWS = 8192
SIZE = 1024
LANES = 16

_info = plsc.get_sparse_core_info()
NC = _info.num_cores
NS = _info.num_subcores
NW = NC * NS
ROWS_PER_W = ROWS // NW
BLK_ROWS = 16
NBLK = ROWS_PER_W // BLK_ROWS
CHUNKS = SIZE // LANES

_mesh = plsc.VectorSubcoreMesh(core_axis_name="c", subcore_axis_name="s")


@functools.partial(
    pl.kernel,
    mesh=_mesh,
    out_type=jax.ShapeDtypeStruct((ROWS, SIZE), jnp.float32),
    scratch_types=[
        pltpu.VMEM((SIZE,), jnp.int32),
        pltpu.VMEM((BLK_ROWS, SIZE), jnp.float32),
        pltpu.VMEM((BLK_ROWS, SIZE), jnp.float32),
        pltpu.VMEM((BLK_ROWS, SIZE), jnp.float32),
        pltpu.VMEM((BLK_ROWS, SIZE), jnp.float32),
        pltpu.SemaphoreType.DMA,
        pltpu.SemaphoreType.DMA,
        pltpu.SemaphoreType.DMA,
        pltpu.SemaphoreType.DMA,
    ],
    compiler_params=pltpu.CompilerParams(needs_layout_passes=False),
)
def _permute_sc(z_hbm, p_hbm, out_hbm, p_v, zbuf0, zbuf1, obuf0, obuf1,
                in_sem0, in_sem1, out_sem0, out_sem1):
    wid = lax.axis_index("s") * NC + lax.axis_index("c")
    base = wid * ROWS_PER_W
    in_sems = (in_sem0, in_sem1)
    out_sems = (out_sem0, out_sem1)
    zbufs = (zbuf0, zbuf1)
    obufs = (obuf0, obuf1)

    pltpu.sync_copy(p_hbm, p_v)

    def start_in(b, s):
        pltpu.async_copy(z_hbm.at[pl.ds(base + b * BLK_ROWS, BLK_ROWS), :],
                         zbufs[s], in_sems[s])

    def wait_in(b, s):
        pltpu.make_async_copy(
            z_hbm.at[pl.ds(base + b * BLK_ROWS, BLK_ROWS), :],
            zbufs[s], in_sems[s]).wait()

    def start_out(b, s):
        pltpu.async_copy(obufs[s],
                         out_hbm.at[pl.ds(base + b * BLK_ROWS, BLK_ROWS), :],
                         out_sems[s])

    def wait_out(b, s):
        pltpu.make_async_copy(
            obufs[s], out_hbm.at[pl.ds(base + b * BLK_ROWS, BLK_ROWS), :],
            out_sems[s]).wait()

    def compute(s):
        zb = zbufs[s]
        ob = obufs[s]

        def jbody(j, _):
            col = j * LANES
            cidx = p_v[pl.ds(col, LANES)]
            rvec = jnp.zeros((LANES,), jnp.int32)
            for r in range(BLK_ROWS):
                val = plsc.load_gather(zb, [rvec, cidx])
                ob[r, pl.ds(col, LANES)] = val
                if r + 1 < BLK_ROWS:
                    rvec = rvec + 1
            return 0

        lax.fori_loop(0, CHUNKS, jbody, 0)

    start_in(0, 0)
    start_in(1, 1)

    def ring(i, _):
        g = i * 2
        for s in range(2):
            b = g + s
            wait_in(b, s)

            @pl.when(i > 0)
            def _():
                wait_out(b - 2, s)

            compute(s)
            start_out(b, s)

            @pl.when(b + 2 < NBLK)
            def _():
                start_in(b + 2, s)

        return 0

    lax.fori_loop(0, NBLK // 2, ring, 0)
    wait_out(NBLK - 2, 0)
    wait_out(NBLK - 1, 1)


def kernel(z, p):
    pi = p.astype(jnp.int32)
    return _permute_sc(z, pi)
